# Initial kernel scaffold; baseline (speedup 1.0000x reference)
import jax
import jax.numpy as jnp
from jax.experimental import pallas as pl


def kernel(x, edge_index, W1, b1, W2, b2):
    raise NotImplementedError("write your pallas kernel here")


# speedup vs baseline: 9.9603x; 9.9603x over previous
import functools

import jax
import jax.numpy as jnp
from jax import lax
from jax.experimental import pallas as pl
from jax.experimental.pallas import tpu as pltpu
from jax.experimental.pallas import tpu_sc as plsc

N = 10000
E = 320000
D1 = 48
D2 = 16
NC, NS = 2, 16
NW = NC * NS
EPW = E // NW
CHUNK = 80
NCH = EPW // CHUNK
RPT = 624
TAIL = N - NS * RPT
TAIL_OFF = NS * RPT


_BLK = 1000


def _proj_body(x_ref, w_ref, c_ref, o_ref):
    o_ref[...] = (
        jnp.dot(x_ref[...], w_ref[...], preferred_element_type=jnp.float32)
        + c_ref[...]
    )


def _project(x, w, col):
    k, d = w.shape
    return pl.pallas_call(
        _proj_body,
        grid=(N // _BLK,),
        in_specs=[
            pl.BlockSpec((_BLK, k), lambda i: (i, 0)),
            pl.BlockSpec((k, d), lambda i: (0, 0)),
            pl.BlockSpec((1, d), lambda i: (0, 0)),
        ],
        out_specs=pl.BlockSpec((_BLK, d), lambda i: (i, 0)),
        out_shape=jax.ShapeDtypeStruct((N, d), jnp.float32),
    )(x, w, col)


def _mid_body(p_ref, parts_ref, b_ref, w_ref, c_ref, o_ref):
    s = parts_ref[0] + parts_ref[1]
    deg = jnp.maximum(s[:, 32:33], 1.0)
    h = jnp.maximum(p_ref[:, :32] + s[:, :32] / deg + b_ref[...], 0.0)
    o_ref[...] = (
        jnp.dot(h, w_ref[...], preferred_element_type=jnp.float32) + c_ref[...]
    )


def _mid(p1, parts1, b1row, w2p, col2):
    return pl.pallas_call(
        _mid_body,
        grid=(N // _BLK,),
        in_specs=[
            pl.BlockSpec((_BLK, D1), lambda i: (i, 0)),
            pl.BlockSpec((NC, _BLK, D1), lambda i: (0, i, 0)),
            pl.BlockSpec((1, 32), lambda i: (0, 0)),
            pl.BlockSpec((32, D2), lambda i: (0, 0)),
            pl.BlockSpec((1, D2), lambda i: (0, 0)),
        ],
        out_specs=pl.BlockSpec((_BLK, D2), lambda i: (i, 0)),
        out_shape=jax.ShapeDtypeStruct((N, D2), jnp.float32),
    )(p1, parts1, b1row, w2p, col2)


def _fin_body(p_ref, parts_ref, b_ref, o_ref):
    s = parts_ref[0] + parts_ref[1]
    deg = jnp.maximum(s[:, 2:3], 1.0)
    o_ref[...] = p_ref[:, :2] + s[:, :2] / deg + b_ref[...]


def _fin(p2, parts2, b2row):
    return pl.pallas_call(
        _fin_body,
        grid=(N // _BLK,),
        in_specs=[
            pl.BlockSpec((_BLK, D2), lambda i: (i, 0)),
            pl.BlockSpec((NC, _BLK, D2), lambda i: (0, i, 0)),
            pl.BlockSpec((1, 2), lambda i: (0, 0)),
        ],
        out_specs=pl.BlockSpec((_BLK, 2), lambda i: (i, 0)),
        out_shape=jax.ShapeDtypeStruct((N, 2), jnp.float32),
    )(p2, parts2, b2row)


def _make_agg(d):
    mesh = plsc.VectorSubcoreMesh(
        core_axis_name="c", subcore_axis_name="s", num_cores=NC, num_subcores=NS
    )

    @functools.partial(
        pl.kernel,
        out_type=jax.ShapeDtypeStruct((NC, N, d), jnp.float32),
        mesh=mesh,
        scratch_types=[
            pltpu.VMEM((NCH, CHUNK), jnp.int32),
            pltpu.VMEM((NCH, CHUNK), jnp.int32),
            pltpu.VMEM((CHUNK, d), jnp.float32),
            pltpu.VMEM_SHARED((N, d), jnp.float32),
            pltpu.SemaphoreType.DMA,
        ],
        compiler_params=pltpu.CompilerParams(use_tc_tiling_on_sc=False),
    )
    def agg(table, src, dst, zeros, out, src_v, dst_v, rows_v, acc, sem):
        c = lax.axis_index("c")
        s = lax.axis_index("s")
        wid = c * NS + s
        pltpu.sync_copy(zeros.at[pl.ds(0, RPT)], acc.at[pl.ds(s * RPT, RPT)])

        @pl.when(s == 0)
        def _():
            pltpu.sync_copy(
                zeros.at[pl.ds(0, TAIL)], acc.at[pl.ds(TAIL_OFF, TAIL)]
            )

        pltpu.sync_copy(src.at[wid], src_v)
        pltpu.sync_copy(dst.at[wid], dst_v)
        plsc.subcore_barrier()

        @pl.loop(0, NCH)
        def _(j):
            pltpu.async_copy(table.at[src_v.at[j]], rows_v, sem).wait()
            pltpu.sync_copy(rows_v, acc.at[dst_v.at[j]], add=True)

        plsc.subcore_barrier()
        pltpu.sync_copy(
            acc.at[pl.ds(s * RPT, RPT)], out.at[c].at[pl.ds(s * RPT, RPT)]
        )

        @pl.when(s == 0)
        def _():
            pltpu.sync_copy(
                acc.at[pl.ds(TAIL_OFF, TAIL)], out.at[c].at[pl.ds(TAIL_OFF, TAIL)]
            )

    return agg


_agg_cache = {}


def _agg(d):
    if d not in _agg_cache:
        _agg_cache[d] = _make_agg(d)
    return _agg_cache[d]


def kernel(x, edge_index, W1, b1, W2, b2):
    src = edge_index[0].reshape(NW, NCH, CHUNK)
    dst = edge_index[1].reshape(NW, NCH, CHUNK)

    w1p = jnp.pad(W1, ((0, 0), (0, D1 - 32)))
    col1 = jnp.zeros((1, D1), jnp.float32).at[0, 32].set(1.0)
    p1 = _project(x, w1p, col1)
    parts1 = _agg(D1)(p1, src, dst, jnp.zeros((RPT, D1), jnp.float32))

    w2p = jnp.pad(W2, ((0, 0), (0, D2 - 2)))
    col2 = jnp.zeros((1, D2), jnp.float32).at[0, 2].set(1.0)
    p2 = _mid(p1, parts1, b1.reshape(1, 32), w2p, col2)
    parts2 = _agg(D2)(p2, src, dst, jnp.zeros((RPT, D2), jnp.float32))

    return _fin(p2, parts2, b2.reshape(1, 2))


# speedup vs baseline: 17.2010x; 1.7270x over previous
import functools

import jax
import jax.numpy as jnp
from jax import lax
from jax.experimental import pallas as pl
from jax.experimental.pallas import tpu as pltpu
from jax.experimental.pallas import tpu_sc as plsc

N = 10000
E = 320000
D1 = 48
D2 = 16
NC, NS = 2, 16
NW = NC * NS
EPW = E // NW
CHUNK = 80
NCH = EPW // CHUNK
NBUF = 5
RPT = 624
TAIL = N - NS * RPT
TAIL_OFF = NS * RPT


_BLK = 1000


def _proj_body(x_ref, w_ref, c_ref, o_ref):
    o_ref[...] = (
        jnp.dot(x_ref[...], w_ref[...], preferred_element_type=jnp.float32)
        + c_ref[...]
    )


def _project(x, w, col):
    k, d = w.shape
    return pl.pallas_call(
        _proj_body,
        grid=(N // _BLK,),
        in_specs=[
            pl.BlockSpec((_BLK, k), lambda i: (i, 0)),
            pl.BlockSpec((k, d), lambda i: (0, 0)),
            pl.BlockSpec((1, d), lambda i: (0, 0)),
        ],
        out_specs=pl.BlockSpec((_BLK, d), lambda i: (i, 0)),
        out_shape=jax.ShapeDtypeStruct((N, d), jnp.float32),
    )(x, w, col)


def _mid_body(p_ref, parts_ref, b_ref, w_ref, c_ref, o_ref):
    s = parts_ref[0] + parts_ref[1]
    deg = jnp.maximum(s[:, 32:33], 1.0)
    h = jnp.maximum(p_ref[:, :32] + s[:, :32] / deg + b_ref[...], 0.0)
    o_ref[...] = (
        jnp.dot(h, w_ref[...], preferred_element_type=jnp.float32) + c_ref[...]
    )


def _mid(p1, parts1, b1row, w2p, col2):
    return pl.pallas_call(
        _mid_body,
        grid=(N // _BLK,),
        in_specs=[
            pl.BlockSpec((_BLK, D1), lambda i: (i, 0)),
            pl.BlockSpec((NC, _BLK, D1), lambda i: (0, i, 0)),
            pl.BlockSpec((1, 32), lambda i: (0, 0)),
            pl.BlockSpec((32, D2), lambda i: (0, 0)),
            pl.BlockSpec((1, D2), lambda i: (0, 0)),
        ],
        out_specs=pl.BlockSpec((_BLK, D2), lambda i: (i, 0)),
        out_shape=jax.ShapeDtypeStruct((N, D2), jnp.float32),
    )(p1, parts1, b1row, w2p, col2)


def _fin_body(p_ref, parts_ref, b_ref, o_ref):
    s = parts_ref[0] + parts_ref[1]
    deg = jnp.maximum(s[:, 2:3], 1.0)
    o_ref[...] = p_ref[:, :2] + s[:, :2] / deg + b_ref[...]


def _fin(p2, parts2, b2row):
    return pl.pallas_call(
        _fin_body,
        grid=(N // _BLK,),
        in_specs=[
            pl.BlockSpec((_BLK, D2), lambda i: (i, 0)),
            pl.BlockSpec((NC, _BLK, D2), lambda i: (0, i, 0)),
            pl.BlockSpec((1, 2), lambda i: (0, 0)),
        ],
        out_specs=pl.BlockSpec((_BLK, 2), lambda i: (i, 0)),
        out_shape=jax.ShapeDtypeStruct((N, 2), jnp.float32),
    )(p2, parts2, b2row)


def _make_agg(d):
    mesh = plsc.VectorSubcoreMesh(
        core_axis_name="c", subcore_axis_name="s", num_cores=NC, num_subcores=NS
    )

    @functools.partial(
        pl.kernel,
        out_type=jax.ShapeDtypeStruct((NC, N, d), jnp.float32),
        mesh=mesh,
        scratch_types=[
            pltpu.VMEM((NCH, CHUNK), jnp.int32),
            pltpu.VMEM((NCH, CHUNK), jnp.int32),
            pltpu.VMEM((NBUF, CHUNK, d), jnp.float32),
            pltpu.VMEM_SHARED((N, d), jnp.float32),
            pltpu.SemaphoreType.DMA((NBUF,)),
            pltpu.SemaphoreType.DMA((NBUF,)),
        ],
        compiler_params=pltpu.CompilerParams(use_tc_tiling_on_sc=False),
    )
    def agg(table, src, dst, zeros, out, src_v, dst_v, rows_v, acc, gsem, ssem):
        c = lax.axis_index("c")
        s = lax.axis_index("s")
        wid = c * NS + s
        pltpu.sync_copy(zeros.at[pl.ds(0, RPT)], acc.at[pl.ds(s * RPT, RPT)])

        @pl.when(s == 0)
        def _():
            pltpu.sync_copy(
                zeros.at[pl.ds(0, TAIL)], acc.at[pl.ds(TAIL_OFF, TAIL)]
            )

        pltpu.sync_copy(src.at[wid], src_v)
        pltpu.sync_copy(dst.at[wid], dst_v)
        plsc.subcore_barrier()

        @pl.loop(0, NCH, step=NBUF)
        def _(j0):
            gds = [
                pltpu.async_copy(
                    table.at[src_v.at[j0 + b]], rows_v.at[b], gsem.at[b]
                )
                for b in range(NBUF)
            ]
            sds = []
            for b in range(NBUF):
                gds[b].wait()
                sds.append(
                    pltpu.async_copy(
                        rows_v.at[b], acc.at[dst_v.at[j0 + b]], ssem.at[b],
                        add=True,
                    )
                )
            for sd in sds:
                sd.wait()

        plsc.subcore_barrier()
        pltpu.sync_copy(
            acc.at[pl.ds(s * RPT, RPT)], out.at[c].at[pl.ds(s * RPT, RPT)]
        )

        @pl.when(s == 0)
        def _():
            pltpu.sync_copy(
                acc.at[pl.ds(TAIL_OFF, TAIL)], out.at[c].at[pl.ds(TAIL_OFF, TAIL)]
            )

    return agg


_agg_cache = {}


def _agg(d):
    if d not in _agg_cache:
        _agg_cache[d] = _make_agg(d)
    return _agg_cache[d]


def kernel(x, edge_index, W1, b1, W2, b2):
    src = edge_index[0].reshape(NW, NCH, CHUNK)
    dst = edge_index[1].reshape(NW, NCH, CHUNK)

    w1p = jnp.pad(W1, ((0, 0), (0, D1 - 32)))
    col1 = jnp.zeros((1, D1), jnp.float32).at[0, 32].set(1.0)
    p1 = _project(x, w1p, col1)
    parts1 = _agg(D1)(p1, src, dst, jnp.zeros((RPT, D1), jnp.float32))

    w2p = jnp.pad(W2, ((0, 0), (0, D2 - 2)))
    col2 = jnp.zeros((1, D2), jnp.float32).at[0, 2].set(1.0)
    p2 = _mid(p1, parts1, b1.reshape(1, 32), w2p, col2)
    parts2 = _agg(D2)(p2, src, dst, jnp.zeros((RPT, D2), jnp.float32))

    return _fin(p2, parts2, b2.reshape(1, 2))


# speedup vs baseline: 19.6975x; 1.1451x over previous
import functools

import jax
import jax.numpy as jnp
from jax import lax
from jax.experimental import pallas as pl
from jax.experimental.pallas import tpu as pltpu
from jax.experimental.pallas import tpu_sc as plsc

N = 10000
E = 320000
D1 = 48
D2 = 16
NC, NS = 2, 16
NW = NC * NS
EPW = E // NW
CHUNK = 80
NCH = EPW // CHUNK
NBUF = 5
RPT = 624
TAIL = N - NS * RPT
TAIL_OFF = NS * RPT


_BLK = 1000


def _proj_body(x_ref, w_ref, c_ref, o_ref):
    o_ref[...] = (
        jnp.dot(x_ref[...], w_ref[...], preferred_element_type=jnp.float32)
        + c_ref[...]
    )


def _project(x, w, col):
    k, d = w.shape
    return pl.pallas_call(
        _proj_body,
        grid=(N // _BLK,),
        in_specs=[
            pl.BlockSpec((_BLK, k), lambda i: (i, 0)),
            pl.BlockSpec((k, d), lambda i: (0, 0)),
            pl.BlockSpec((1, d), lambda i: (0, 0)),
        ],
        out_specs=pl.BlockSpec((_BLK, d), lambda i: (i, 0)),
        out_shape=jax.ShapeDtypeStruct((N, d), jnp.float32),
    )(x, w, col)


def _mid_body(p_ref, parts_ref, b_ref, w_ref, c_ref, o_ref):
    s = parts_ref[0] + parts_ref[1]
    deg = jnp.maximum(s[:, 32:33], 1.0)
    h = jnp.maximum(p_ref[:, :32] + s[:, :32] / deg + b_ref[...], 0.0)
    o_ref[...] = (
        jnp.dot(h, w_ref[...], preferred_element_type=jnp.float32) + c_ref[...]
    )


def _mid(p1, parts1, b1row, w2p, col2):
    return pl.pallas_call(
        _mid_body,
        grid=(N // _BLK,),
        in_specs=[
            pl.BlockSpec((_BLK, D1), lambda i: (i, 0)),
            pl.BlockSpec((NC, _BLK, D1), lambda i: (0, i, 0)),
            pl.BlockSpec((1, 32), lambda i: (0, 0)),
            pl.BlockSpec((32, D2), lambda i: (0, 0)),
            pl.BlockSpec((1, D2), lambda i: (0, 0)),
        ],
        out_specs=pl.BlockSpec((_BLK, D2), lambda i: (i, 0)),
        out_shape=jax.ShapeDtypeStruct((N, D2), jnp.float32),
    )(p1, parts1, b1row, w2p, col2)


def _fin_body(p_ref, parts_ref, b_ref, o_ref):
    s = parts_ref[0] + parts_ref[1]
    deg = jnp.maximum(s[:, 2:3], 1.0)
    o_ref[...] = p_ref[:, :2] + s[:, :2] / deg + b_ref[...]


def _fin(p2, parts2, b2row):
    return pl.pallas_call(
        _fin_body,
        grid=(N // _BLK,),
        in_specs=[
            pl.BlockSpec((_BLK, D2), lambda i: (i, 0)),
            pl.BlockSpec((NC, _BLK, D2), lambda i: (0, i, 0)),
            pl.BlockSpec((1, 2), lambda i: (0, 0)),
        ],
        out_specs=pl.BlockSpec((_BLK, 2), lambda i: (i, 0)),
        out_shape=jax.ShapeDtypeStruct((N, 2), jnp.float32),
    )(p2, parts2, b2row)


def _make_agg(d):
    mesh = plsc.VectorSubcoreMesh(
        core_axis_name="c", subcore_axis_name="s", num_cores=NC, num_subcores=NS
    )

    @functools.partial(
        pl.kernel,
        out_type=jax.ShapeDtypeStruct((NC, N, d), jnp.float32),
        mesh=mesh,
        scratch_types=[
            pltpu.VMEM((NCH, CHUNK), jnp.int32),
            pltpu.VMEM((NCH, CHUNK), jnp.int32),
            pltpu.VMEM((2 * NBUF, CHUNK, d), jnp.float32),
            pltpu.VMEM_SHARED((N, d), jnp.float32),
            pltpu.SemaphoreType.DMA((2 * NBUF,)),
            pltpu.SemaphoreType.DMA((2 * NBUF,)),
        ],
        compiler_params=pltpu.CompilerParams(use_tc_tiling_on_sc=False),
    )
    def agg(table, src, dst, zeros, out, src_v, dst_v, rows_v, acc, gsem, ssem):
        c = lax.axis_index("c")
        s = lax.axis_index("s")
        wid = c * NS + s
        pltpu.sync_copy(zeros.at[pl.ds(0, RPT)], acc.at[pl.ds(s * RPT, RPT)])

        @pl.when(s == 0)
        def _():
            pltpu.sync_copy(
                zeros.at[pl.ds(0, TAIL)], acc.at[pl.ds(TAIL_OFF, TAIL)]
            )

        pltpu.sync_copy(src.at[wid], src_v)
        pltpu.sync_copy(dst.at[wid], dst_v)
        plsc.subcore_barrier()

        def fire_gather(j, b):
            pltpu.async_copy(table.at[src_v.at[j]], rows_v.at[b], gsem.at[b])

        def wait_gather(j, b):
            pltpu.make_async_copy(
                table.at[src_v.at[j]], rows_v.at[b], gsem.at[b]
            ).wait()

        def fire_scatter(j, b):
            pltpu.async_copy(
                rows_v.at[b], acc.at[dst_v.at[j]], ssem.at[b], add=True
            )

        def wait_scatter(j, b):
            pltpu.make_async_copy(
                rows_v.at[b], acc.at[dst_v.at[j]], ssem.at[b]
            ).wait()

        def drain_set(j0, off):
            for b in range(NBUF):
                wait_gather(j0 + off + b, off + b)
                fire_scatter(j0 + off + b, off + b)
            for b in range(NBUF):
                wait_scatter(j0 + off + b, off + b)

        for b in range(NBUF):
            fire_gather(b, b)

        @pl.loop(0, NCH - 2 * NBUF, step=2 * NBUF)
        def _(j0):
            for b in range(NBUF):
                fire_gather(j0 + NBUF + b, NBUF + b)
            drain_set(j0, 0)
            for b in range(NBUF):
                fire_gather(j0 + 2 * NBUF + b, b)
            drain_set(j0, NBUF)

        j_tail = NCH - NBUF
        drain_set(j_tail, 0)

        plsc.subcore_barrier()
        pltpu.sync_copy(
            acc.at[pl.ds(s * RPT, RPT)], out.at[c].at[pl.ds(s * RPT, RPT)]
        )

        @pl.when(s == 0)
        def _():
            pltpu.sync_copy(
                acc.at[pl.ds(TAIL_OFF, TAIL)], out.at[c].at[pl.ds(TAIL_OFF, TAIL)]
            )

    return agg


_agg_cache = {}


def _agg(d):
    if d not in _agg_cache:
        _agg_cache[d] = _make_agg(d)
    return _agg_cache[d]


def kernel(x, edge_index, W1, b1, W2, b2):
    src = edge_index[0].reshape(NW, NCH, CHUNK)
    dst = edge_index[1].reshape(NW, NCH, CHUNK)

    w1p = jnp.pad(W1, ((0, 0), (0, D1 - 32)))
    col1 = jnp.zeros((1, D1), jnp.float32).at[0, 32].set(1.0)
    p1 = _project(x, w1p, col1)
    parts1 = _agg(D1)(p1, src, dst, jnp.zeros((RPT, D1), jnp.float32))

    w2p = jnp.pad(W2, ((0, 0), (0, D2 - 2)))
    col2 = jnp.zeros((1, D2), jnp.float32).at[0, 2].set(1.0)
    p2 = _mid(p1, parts1, b1.reshape(1, 32), w2p, col2)
    parts2 = _agg(D2)(p2, src, dst, jnp.zeros((RPT, D2), jnp.float32))

    return _fin(p2, parts2, b2.reshape(1, 2))


# speedup vs baseline: 20.4158x; 1.0365x over previous
import functools

import jax
import jax.numpy as jnp
from jax import lax
from jax.experimental import pallas as pl
from jax.experimental.pallas import tpu as pltpu
from jax.experimental.pallas import tpu_sc as plsc

N = 10000
E = 320000
D1 = 48
D2 = 16
NC, NS = 2, 16
NW = NC * NS
EPW = E // NW
CHUNK = 125
NCH = EPW // CHUNK
NBUF = 5
RPT = 624
TAIL = N - NS * RPT
TAIL_OFF = NS * RPT


_BLK = 2000


def _proj_body(x_ref, w_ref, c_ref, o_ref):
    o_ref[...] = (
        jnp.dot(x_ref[...], w_ref[...], preferred_element_type=jnp.float32)
        + c_ref[...]
    )


def _project(x, w, col):
    k, d = w.shape
    return pl.pallas_call(
        _proj_body,
        grid=(N // _BLK,),
        in_specs=[
            pl.BlockSpec((_BLK, k), lambda i: (i, 0)),
            pl.BlockSpec((k, d), lambda i: (0, 0)),
            pl.BlockSpec((1, d), lambda i: (0, 0)),
        ],
        out_specs=pl.BlockSpec((_BLK, d), lambda i: (i, 0)),
        out_shape=jax.ShapeDtypeStruct((N, d), jnp.float32),
    )(x, w, col)


def _mid_body(p_ref, parts_ref, b_ref, w_ref, c_ref, o_ref):
    s = parts_ref[0] + parts_ref[1]
    deg = jnp.maximum(s[:, 32:33], 1.0)
    h = jnp.maximum(p_ref[:, :32] + s[:, :32] / deg + b_ref[...], 0.0)
    o_ref[...] = (
        jnp.dot(h, w_ref[...], preferred_element_type=jnp.float32) + c_ref[...]
    )


def _mid(p1, parts1, b1row, w2p, col2):
    return pl.pallas_call(
        _mid_body,
        grid=(N // _BLK,),
        in_specs=[
            pl.BlockSpec((_BLK, D1), lambda i: (i, 0)),
            pl.BlockSpec((NC, _BLK, D1), lambda i: (0, i, 0)),
            pl.BlockSpec((1, 32), lambda i: (0, 0)),
            pl.BlockSpec((32, D2), lambda i: (0, 0)),
            pl.BlockSpec((1, D2), lambda i: (0, 0)),
        ],
        out_specs=pl.BlockSpec((_BLK, D2), lambda i: (i, 0)),
        out_shape=jax.ShapeDtypeStruct((N, D2), jnp.float32),
    )(p1, parts1, b1row, w2p, col2)


def _fin_body(p_ref, parts_ref, b_ref, o_ref):
    s = parts_ref[0] + parts_ref[1]
    deg = jnp.maximum(s[:, 2:3], 1.0)
    o_ref[...] = p_ref[:, :2] + s[:, :2] / deg + b_ref[...]


def _fin(p2, parts2, b2row):
    return pl.pallas_call(
        _fin_body,
        grid=(N // _BLK,),
        in_specs=[
            pl.BlockSpec((_BLK, D2), lambda i: (i, 0)),
            pl.BlockSpec((NC, _BLK, D2), lambda i: (0, i, 0)),
            pl.BlockSpec((1, 2), lambda i: (0, 0)),
        ],
        out_specs=pl.BlockSpec((_BLK, 2), lambda i: (i, 0)),
        out_shape=jax.ShapeDtypeStruct((N, 2), jnp.float32),
    )(p2, parts2, b2row)


def _make_agg(d):
    mesh = plsc.VectorSubcoreMesh(
        core_axis_name="c", subcore_axis_name="s", num_cores=NC, num_subcores=NS
    )

    @functools.partial(
        pl.kernel,
        out_type=jax.ShapeDtypeStruct((NC, N, d), jnp.float32),
        mesh=mesh,
        scratch_types=[
            pltpu.VMEM((NCH, CHUNK), jnp.int32),
            pltpu.VMEM((NCH, CHUNK), jnp.int32),
            pltpu.VMEM((2 * NBUF, CHUNK, d), jnp.float32),
            pltpu.VMEM_SHARED((N, d), jnp.float32),
            pltpu.SemaphoreType.DMA((2 * NBUF,)),
            pltpu.SemaphoreType.DMA((2 * NBUF,)),
        ],
        compiler_params=pltpu.CompilerParams(use_tc_tiling_on_sc=False),
    )
    def agg(table, src, dst, zeros, out, src_v, dst_v, rows_v, acc, gsem, ssem):
        c = lax.axis_index("c")
        s = lax.axis_index("s")
        wid = c * NS + s
        pltpu.sync_copy(zeros.at[pl.ds(0, RPT)], acc.at[pl.ds(s * RPT, RPT)])

        @pl.when(s == 0)
        def _():
            pltpu.sync_copy(
                zeros.at[pl.ds(0, TAIL)], acc.at[pl.ds(TAIL_OFF, TAIL)]
            )

        pltpu.sync_copy(src.at[wid], src_v)
        pltpu.sync_copy(dst.at[wid], dst_v)
        plsc.subcore_barrier()

        def fire_gather(j, b):
            pltpu.async_copy(table.at[src_v.at[j]], rows_v.at[b], gsem.at[b])

        def wait_gather(j, b):
            pltpu.make_async_copy(
                table.at[src_v.at[j]], rows_v.at[b], gsem.at[b]
            ).wait()

        def fire_scatter(j, b):
            pltpu.async_copy(
                rows_v.at[b], acc.at[dst_v.at[j]], ssem.at[b], add=True
            )

        def wait_scatter(j, b):
            pltpu.make_async_copy(
                rows_v.at[b], acc.at[dst_v.at[j]], ssem.at[b]
            ).wait()

        def drain_set(j0, off):
            for b in range(NBUF):
                wait_gather(j0 + off + b, off + b)
                fire_scatter(j0 + off + b, off + b)
            for b in range(NBUF):
                wait_scatter(j0 + off + b, off + b)

        for b in range(NBUF):
            fire_gather(b, b)

        @pl.loop(0, NCH - 2 * NBUF, step=2 * NBUF)
        def _(j0):
            for b in range(NBUF):
                fire_gather(j0 + NBUF + b, NBUF + b)
            drain_set(j0, 0)
            for b in range(NBUF):
                fire_gather(j0 + 2 * NBUF + b, b)
            drain_set(j0, NBUF)

        j_tail = NCH - 2 * NBUF
        for b in range(NBUF):
            fire_gather(j_tail + NBUF + b, NBUF + b)
        drain_set(j_tail, 0)
        drain_set(j_tail, NBUF)

        plsc.subcore_barrier()
        pltpu.sync_copy(
            acc.at[pl.ds(s * RPT, RPT)], out.at[c].at[pl.ds(s * RPT, RPT)]
        )

        @pl.when(s == 0)
        def _():
            pltpu.sync_copy(
                acc.at[pl.ds(TAIL_OFF, TAIL)], out.at[c].at[pl.ds(TAIL_OFF, TAIL)]
            )

    return agg


_agg_cache = {}


def _agg(d):
    if d not in _agg_cache:
        _agg_cache[d] = _make_agg(d)
    return _agg_cache[d]


def kernel(x, edge_index, W1, b1, W2, b2):
    src = edge_index[0].reshape(NW, NCH, CHUNK)
    dst = edge_index[1].reshape(NW, NCH, CHUNK)

    w1p = jnp.pad(W1, ((0, 0), (0, D1 - 32)))
    col1 = jnp.zeros((1, D1), jnp.float32).at[0, 32].set(1.0)
    p1 = _project(x, w1p, col1)
    parts1 = _agg(D1)(p1, src, dst, jnp.zeros((RPT, D1), jnp.float32))

    w2p = jnp.pad(W2, ((0, 0), (0, D2 - 2)))
    col2 = jnp.zeros((1, D2), jnp.float32).at[0, 2].set(1.0)
    p2 = _mid(p1, parts1, b1.reshape(1, 32), w2p, col2)
    parts2 = _agg(D2)(p2, src, dst, jnp.zeros((RPT, D2), jnp.float32))

    return _fin(p2, parts2, b2.reshape(1, 2))


# speedup vs baseline: 22.5059x; 1.1024x over previous
import functools

import jax
import jax.numpy as jnp
from jax import lax
from jax.experimental import pallas as pl
from jax.experimental.pallas import tpu as pltpu
from jax.experimental.pallas import tpu_sc as plsc

N = 10000
E = 320000
D1 = 48
D2 = 16
NC, NS = 2, 16
NW = NC * NS
EPW = E // NW
CHUNK = 125
NCH = EPW // CHUNK
NBUF = 5
RPT = 624
TAIL = N - NS * RPT
TAIL_OFF = NS * RPT


_BLK = 2000


def _proj_body(x_ref, w_ref, o_ref):
    dw = w_ref.shape[1]
    o_ref[:, :dw] = jnp.dot(
        x_ref[...], w_ref[...], preferred_element_type=jnp.float32
    )
    o_ref[:, dw:dw + 1] = jnp.ones((_BLK, 1), jnp.float32)
    o_ref[:, dw + 1:] = jnp.zeros((_BLK, o_ref.shape[1] - dw - 1), jnp.float32)


def _project(x, w, d):
    k, dw = w.shape
    return pl.pallas_call(
        _proj_body,
        grid=(N // _BLK,),
        in_specs=[
            pl.BlockSpec((_BLK, k), lambda i: (i, 0)),
            pl.BlockSpec((k, dw), lambda i: (0, 0)),
        ],
        out_specs=pl.BlockSpec((_BLK, d), lambda i: (i, 0)),
        out_shape=jax.ShapeDtypeStruct((N, d), jnp.float32),
    )(x, w)


def _mid_body(p_ref, parts_ref, b_ref, w_ref, o_ref):
    s = parts_ref[0] + parts_ref[1]
    deg = jnp.maximum(s[:, 32:33], 1.0)
    h = jnp.maximum(p_ref[:, :32] + s[:, :32] / deg + b_ref[...], 0.0)
    o_ref[:, :2] = jnp.dot(h, w_ref[...], preferred_element_type=jnp.float32)
    o_ref[:, 2:3] = jnp.ones((_BLK, 1), jnp.float32)
    o_ref[:, 3:] = jnp.zeros((_BLK, D2 - 3), jnp.float32)


def _mid(p1, parts1, b1row, w2):
    return pl.pallas_call(
        _mid_body,
        grid=(N // _BLK,),
        in_specs=[
            pl.BlockSpec((_BLK, D1), lambda i: (i, 0)),
            pl.BlockSpec((NC, _BLK, D1), lambda i: (0, i, 0)),
            pl.BlockSpec((1, 32), lambda i: (0, 0)),
            pl.BlockSpec((32, 2), lambda i: (0, 0)),
        ],
        out_specs=pl.BlockSpec((_BLK, D2), lambda i: (i, 0)),
        out_shape=jax.ShapeDtypeStruct((N, D2), jnp.float32),
    )(p1, parts1, b1row, w2)


def _fin_body(p_ref, parts_ref, b_ref, o_ref):
    s = parts_ref[0] + parts_ref[1]
    deg = jnp.maximum(s[:, 2:3], 1.0)
    o_ref[...] = p_ref[:, :2] + s[:, :2] / deg + b_ref[...]


def _fin(p2, parts2, b2row):
    return pl.pallas_call(
        _fin_body,
        grid=(N // _BLK,),
        in_specs=[
            pl.BlockSpec((_BLK, D2), lambda i: (i, 0)),
            pl.BlockSpec((NC, _BLK, D2), lambda i: (0, i, 0)),
            pl.BlockSpec((1, 2), lambda i: (0, 0)),
        ],
        out_specs=pl.BlockSpec((_BLK, 2), lambda i: (i, 0)),
        out_shape=jax.ShapeDtypeStruct((N, 2), jnp.float32),
    )(p2, parts2, b2row)


def _make_agg(d):
    mesh = plsc.VectorSubcoreMesh(
        core_axis_name="c", subcore_axis_name="s", num_cores=NC, num_subcores=NS
    )

    @functools.partial(
        pl.kernel,
        out_type=jax.ShapeDtypeStruct((NC, N, d), jnp.float32),
        mesh=mesh,
        scratch_types=[
            pltpu.VMEM((NCH, CHUNK), jnp.int32),
            pltpu.VMEM((NCH, CHUNK), jnp.int32),
            pltpu.VMEM((2 * NBUF, CHUNK, d), jnp.float32),
            pltpu.VMEM_SHARED((N, d), jnp.float32),
            pltpu.SemaphoreType.DMA((2 * NBUF,)),
            pltpu.SemaphoreType.DMA((2 * NBUF,)),
        ],
        compiler_params=pltpu.CompilerParams(use_tc_tiling_on_sc=False),
    )
    def agg(table, edges, zeros, out, src_v, dst_v, rows_v, acc, gsem, ssem):
        c = lax.axis_index("c")
        s = lax.axis_index("s")
        wid = c * NS + s
        pltpu.sync_copy(zeros.at[pl.ds(0, RPT)], acc.at[pl.ds(s * RPT, RPT)])

        @pl.when(s == 0)
        def _():
            pltpu.sync_copy(
                zeros.at[pl.ds(0, TAIL)], acc.at[pl.ds(TAIL_OFF, TAIL)]
            )

        pltpu.sync_copy(edges.at[0, wid], src_v)
        pltpu.sync_copy(edges.at[1, wid], dst_v)
        plsc.subcore_barrier()

        def fire_gather(j, b):
            pltpu.async_copy(table.at[src_v.at[j]], rows_v.at[b], gsem.at[b])

        def wait_gather(j, b):
            pltpu.make_async_copy(
                table.at[src_v.at[j]], rows_v.at[b], gsem.at[b]
            ).wait()

        def fire_scatter(j, b):
            pltpu.async_copy(
                rows_v.at[b], acc.at[dst_v.at[j]], ssem.at[b], add=True
            )

        def wait_scatter(j, b):
            pltpu.make_async_copy(
                rows_v.at[b], acc.at[dst_v.at[j]], ssem.at[b]
            ).wait()

        def drain_set(j0, off):
            for b in range(NBUF):
                wait_gather(j0 + off + b, off + b)
                fire_scatter(j0 + off + b, off + b)
            for b in range(NBUF):
                wait_scatter(j0 + off + b, off + b)

        for b in range(NBUF):
            fire_gather(b, b)

        @pl.loop(0, NCH - 2 * NBUF, step=2 * NBUF)
        def _(j0):
            for b in range(NBUF):
                fire_gather(j0 + NBUF + b, NBUF + b)
            drain_set(j0, 0)
            for b in range(NBUF):
                fire_gather(j0 + 2 * NBUF + b, b)
            drain_set(j0, NBUF)

        j_tail = NCH - 2 * NBUF
        for b in range(NBUF):
            fire_gather(j_tail + NBUF + b, NBUF + b)
        drain_set(j_tail, 0)
        drain_set(j_tail, NBUF)

        plsc.subcore_barrier()
        pltpu.sync_copy(
            acc.at[pl.ds(s * RPT, RPT)], out.at[c].at[pl.ds(s * RPT, RPT)]
        )

        @pl.when(s == 0)
        def _():
            pltpu.sync_copy(
                acc.at[pl.ds(TAIL_OFF, TAIL)], out.at[c].at[pl.ds(TAIL_OFF, TAIL)]
            )

    return agg


_agg_cache = {}


def _agg(d):
    if d not in _agg_cache:
        _agg_cache[d] = _make_agg(d)
    return _agg_cache[d]


def kernel(x, edge_index, W1, b1, W2, b2):
    edges = edge_index.reshape(2, NW, NCH, CHUNK)

    p1 = _project(x, W1, D1)
    parts1 = _agg(D1)(p1, edges, jnp.zeros((RPT, D1), jnp.float32))

    p2 = _mid(p1, parts1, b1.reshape(1, 32), W2)
    parts2 = _agg(D2)(p2, edges, jnp.zeros((RPT, D2), jnp.float32))

    return _fin(p2, parts2, b2.reshape(1, 2))


# speedup vs baseline: 22.5279x; 1.0010x over previous
import functools

import jax
import jax.numpy as jnp
from jax import lax
from jax.experimental import pallas as pl
from jax.experimental.pallas import tpu as pltpu
from jax.experimental.pallas import tpu_sc as plsc

N = 10000
E = 320000
D1 = 32
D2 = 16
NC, NS = 2, 16
NW = NC * NS
EPW = E // NW
NBUF = 5
RPT = 624
TAIL = N - NS * RPT
TAIL_OFF = NS * RPT
DROWS = 625
DRPT = DROWS // NS


_BLK = 2000


def _proj_body(x_ref, w_ref, o_ref):
    o_ref[...] = jnp.dot(
        x_ref[...], w_ref[...], preferred_element_type=jnp.float32
    )


def _project(x, w):
    k, d = w.shape
    return pl.pallas_call(
        _proj_body,
        grid=(N // _BLK,),
        in_specs=[
            pl.BlockSpec((_BLK, k), lambda i: (i, 0)),
            pl.BlockSpec((k, d), lambda i: (0, 0)),
        ],
        out_specs=pl.BlockSpec((_BLK, d), lambda i: (i, 0)),
        out_shape=jax.ShapeDtypeStruct((N, d), jnp.float32),
    )(x, w)


def _mid_body(p_ref, parts_ref, b_ref, w_ref, o_ref):
    h = jnp.maximum(p_ref[...] + parts_ref[0] + parts_ref[1] + b_ref[...], 0.0)
    o_ref[:, :2] = jnp.dot(h, w_ref[...], preferred_element_type=jnp.float32)
    o_ref[:, 2:3] = jnp.ones((_BLK, 1), jnp.float32)
    o_ref[:, 3:] = jnp.zeros((_BLK, D2 - 3), jnp.float32)


def _mid(p1, parts1, b1row, w2):
    return pl.pallas_call(
        _mid_body,
        grid=(N // _BLK,),
        in_specs=[
            pl.BlockSpec((_BLK, D1), lambda i: (i, 0)),
            pl.BlockSpec((NC, _BLK, D1), lambda i: (0, i, 0)),
            pl.BlockSpec((1, 32), lambda i: (0, 0)),
            pl.BlockSpec((32, 2), lambda i: (0, 0)),
        ],
        out_specs=pl.BlockSpec((_BLK, D2), lambda i: (i, 0)),
        out_shape=jax.ShapeDtypeStruct((N, D2), jnp.float32),
    )(p1, parts1, b1row, w2)


def _fin_body(p_ref, parts_ref, b_ref, o_ref):
    s = parts_ref[0] + parts_ref[1]
    deg = jnp.maximum(s[:, 2:3], 1.0)
    o_ref[...] = p_ref[:, :2] + s[:, :2] / deg + b_ref[...]


def _fin(p2, parts2, b2row):
    return pl.pallas_call(
        _fin_body,
        grid=(N // _BLK,),
        in_specs=[
            pl.BlockSpec((_BLK, D2), lambda i: (i, 0)),
            pl.BlockSpec((NC, _BLK, D2), lambda i: (0, i, 0)),
            pl.BlockSpec((1, 2), lambda i: (0, 0)),
        ],
        out_specs=pl.BlockSpec((_BLK, 2), lambda i: (i, 0)),
        out_shape=jax.ShapeDtypeStruct((N, 2), jnp.float32),
    )(p2, parts2, b2row)


def _lane_bcast(v, lane):
    idx = jnp.broadcast_to(lane, (16, 1)).astype(jnp.int32)
    return lax.gather(
        v,
        idx,
        lax.GatherDimensionNumbers(
            offset_dims=(), collapsed_slice_dims=(0,), start_index_map=(0,)
        ),
        (1,),
        mode=lax.GatherScatterMode.PROMISE_IN_BOUNDS,
    )


def _make_agg(d, chunk, first):
    nch = EPW // chunk
    mesh = plsc.VectorSubcoreMesh(
        core_axis_name="c", subcore_axis_name="s", num_cores=NC, num_subcores=NS
    )

    scratch = {
        "src_v": pltpu.VMEM((nch, chunk), jnp.int32),
        "dst_v": pltpu.VMEM((nch, chunk), jnp.int32),
        "rows_v": pltpu.VMEM((2 * NBUF, chunk, d), jnp.float32),
        "acc": pltpu.VMEM_SHARED((N, d), jnp.float32),
        "gsem": pltpu.SemaphoreType.DMA((2 * NBUF,)),
        "ssem": pltpu.SemaphoreType.DMA((2 * NBUF,)),
    }
    if first:
        scratch.update(
            dst2_v=pltpu.VMEM((nch, chunk), jnp.int32),
            deg_v=pltpu.VMEM((DROWS, 16), jnp.float32),
            degsp=pltpu.VMEM_SHARED((NS, DROWS, 16), jnp.float32),
            degsl_v=pltpu.VMEM((NS, DRPT + 1, 16), jnp.float32),
            invd_v=pltpu.VMEM((DRPT + 1, 16), jnp.float32),
            work_v=pltpu.VMEM((RPT, d), jnp.float32),
        )

    @functools.partial(
        pl.kernel,
        out_type=jax.ShapeDtypeStruct((NC, N, d), jnp.float32),
        mesh=mesh,
        scratch_types=scratch,
        compiler_params=pltpu.CompilerParams(
            use_tc_tiling_on_sc=False, needs_layout_passes=not first
        ),
    )
    def agg(table, edges, zeros, out, src_v, dst_v, rows_v, acc, gsem, ssem,
            dst2_v=None, deg_v=None, degsp=None, degsl_v=None, invd_v=None,
            work_v=None):
        c = lax.axis_index("c")
        s = lax.axis_index("s")
        wid = c * NS + s
        pltpu.sync_copy(zeros.at[pl.ds(0, RPT)], acc.at[pl.ds(s * RPT, RPT)])

        @pl.when(s == 0)
        def _():
            pltpu.sync_copy(
                zeros.at[pl.ds(0, TAIL)], acc.at[pl.ds(TAIL_OFF, TAIL)]
            )

        pltpu.sync_copy(edges.at[0, wid], src_v)
        pltpu.sync_copy(edges.at[1, wid], dst_v)
        if first:
            wid2 = (1 - c) * NS + s
            pltpu.sync_copy(edges.at[1, wid2], dst2_v)
            ones16 = jnp.ones((16,), jnp.float32)

            @pl.loop(0, DROWS)
            def _(i):
                deg_v[i] = jnp.zeros((16,), jnp.float32)

        plsc.subcore_barrier()

        def deg_chunk(j):
            for ref in (dst_v, dst2_v):
                for q in range(chunk // 16):
                    dd = ref[j, pl.ds(q * 16, 16)]
                    hi = lax.shift_right_logical(dd, 4)
                    lo = lax.bitwise_and(dd, 15)
                    plsc.addupdate_scatter(deg_v, [hi, lo], ones16)

        def fire_gather(j, b):
            pltpu.async_copy(table.at[src_v.at[j]], rows_v.at[b], gsem.at[b])

        def wait_gather(j, b):
            pltpu.make_async_copy(
                table.at[src_v.at[j]], rows_v.at[b], gsem.at[b]
            ).wait()

        def fire_scatter(j, b):
            pltpu.async_copy(
                rows_v.at[b], acc.at[dst_v.at[j]], ssem.at[b], add=True
            )

        def wait_scatter(j, b):
            pltpu.make_async_copy(
                rows_v.at[b], acc.at[dst_v.at[j]], ssem.at[b]
            ).wait()

        def drain_set(j0, off):
            for b in range(NBUF):
                wait_gather(j0 + off + b, off + b)
                fire_scatter(j0 + off + b, off + b)
                if first:
                    deg_chunk(j0 + off + b)
            for b in range(NBUF):
                wait_scatter(j0 + off + b, off + b)

        for b in range(NBUF):
            fire_gather(b, b)

        if nch % (2 * NBUF) == NBUF:
            @pl.loop(0, nch - 2 * NBUF, step=2 * NBUF)
            def _(j0):
                for b in range(NBUF):
                    fire_gather(j0 + NBUF + b, NBUF + b)
                drain_set(j0, 0)
                for b in range(NBUF):
                    fire_gather(j0 + 2 * NBUF + b, b)
                drain_set(j0, NBUF)

            drain_set(nch - NBUF, 0)
        else:
            assert nch % (2 * NBUF) == 0

            @pl.loop(0, nch - 2 * NBUF, step=2 * NBUF)
            def _(j0):
                for b in range(NBUF):
                    fire_gather(j0 + NBUF + b, NBUF + b)
                drain_set(j0, 0)
                for b in range(NBUF):
                    fire_gather(j0 + 2 * NBUF + b, b)
                drain_set(j0, NBUF)

            j_tail = nch - 2 * NBUF
            for b in range(NBUF):
                fire_gather(j_tail + NBUF + b, NBUF + b)
            drain_set(j_tail, 0)
            drain_set(j_tail, NBUF)

        plsc.subcore_barrier()

        if not first:
            pltpu.sync_copy(
                acc.at[pl.ds(s * RPT, RPT)], out.at[c].at[pl.ds(s * RPT, RPT)]
            )

            @pl.when(s == 0)
            def _():
                pltpu.sync_copy(
                    acc.at[pl.ds(TAIL_OFF, TAIL)],
                    out.at[c].at[pl.ds(TAIL_OFF, TAIL)],
                )
        else:
            pltpu.sync_copy(deg_v, degsp.at[s])
            plsc.subcore_barrier()
            r0 = s * DRPT
            for t in range(NS):
                pltpu.sync_copy(
                    degsp.at[t].at[pl.ds(r0, DRPT)],
                    degsl_v.at[t].at[pl.ds(0, DRPT)],
                )

            @pl.when(s == 0)
            def _():
                for t in range(NS):
                    pltpu.sync_copy(
                        degsp.at[t].at[pl.ds(DROWS - 1, 1)],
                        degsl_v.at[t].at[pl.ds(DRPT, 1)],
                    )

            @pl.loop(0, DRPT + 1)
            def _(r):
                tot = degsl_v[0, r]
                for t in range(1, NS):
                    tot = tot + degsl_v[t, r]
                invd_v[r] = 1.0 / jnp.maximum(tot, 1.0)

            pltpu.sync_copy(acc.at[pl.ds(s * RPT, RPT)], work_v)

            @pl.loop(0, RPT)
            def _(i):
                bc = _lane_bcast(invd_v[lax.shift_right_logical(i, 4)],
                                 lax.bitwise_and(i, 15))
                for q in range(d // 16):
                    work_v[i, pl.ds(q * 16, 16)] = (
                        work_v[i, pl.ds(q * 16, 16)] * bc
                    )

            pltpu.sync_copy(work_v, out.at[c].at[pl.ds(s * RPT, RPT)])

            @pl.when(s == 0)
            def _():
                pltpu.sync_copy(
                    acc.at[pl.ds(TAIL_OFF, TAIL)], work_v.at[pl.ds(0, TAIL)]
                )

                @pl.loop(0, TAIL)
                def _(i):
                    bc = _lane_bcast(invd_v[DRPT], i)
                    for q in range(d // 16):
                        work_v[i, pl.ds(q * 16, 16)] = (
                            work_v[i, pl.ds(q * 16, 16)] * bc
                        )

                pltpu.sync_copy(
                    work_v.at[pl.ds(0, TAIL)],
                    out.at[c].at[pl.ds(TAIL_OFF, TAIL)],
                )

    return agg


_agg_cache = {}


def _agg(d, chunk, first):
    key = (d, chunk, first)
    if key not in _agg_cache:
        _agg_cache[key] = _make_agg(d, chunk, first)
    return _agg_cache[key]


def kernel(x, edge_index, W1, b1, W2, b2):
    p1 = _project(x, W1)
    edges = edge_index.reshape(2, NW, EPW // 80, 80)
    parts1 = _agg(D1, 80, True)(
        p1, edges, jnp.zeros((RPT, D1), jnp.float32)
    )

    p2 = _mid(p1, parts1, b1.reshape(1, 32), W2)
    parts2 = _agg(D2, 80, False)(
        p2, edges, jnp.zeros((RPT, D2), jnp.float32)
    )

    return _fin(p2, parts2, b2.reshape(1, 2))
